# baseline (speedup 1.0000x reference)
import functools

import jax
import jax.numpy as jnp
from jax import lax
from jax.experimental import pallas as pl
from jax.experimental.pallas import tpu as pltpu
from jax.experimental.pallas import tpu_sc as plsc

NC = 2
NS = 16
NW = NC * NS


def _ln_relu_body(x_ref, g_ref, bt_ref, bias_ref, wr_ref, h_ref, r_ref):
    xv = x_ref[...]
    mu = jnp.mean(xv, axis=1, keepdims=True)
    var = jnp.mean((xv - mu) ** 2, axis=1, keepdims=True)
    h = (xv - mu) * lax.rsqrt(var + 1e-5) * g_ref[...] + bt_ref[...]
    h = jnp.maximum(h, 0.0)
    h_ref[...] = h
    r_ref[...] = xv + bias_ref[...] + jnp.dot(
        h, wr_ref[...], preferred_element_type=jnp.float32)


def _ln_relu_root(x, gamma, beta, w_root, bias, blk):
    n, d = x.shape
    return pl.pallas_call(
        _ln_relu_body,
        grid=(n // blk,),
        in_specs=[
            pl.BlockSpec((blk, d), lambda i: (i, 0)),
            pl.BlockSpec((1, d), lambda i: (0, 0)),
            pl.BlockSpec((1, d), lambda i: (0, 0)),
            pl.BlockSpec((1, d), lambda i: (0, 0)),
            pl.BlockSpec((d, d), lambda i: (0, 0)),
        ],
        out_specs=[
            pl.BlockSpec((blk, d), lambda i: (i, 0)),
            pl.BlockSpec((blk, d), lambda i: (i, 0)),
        ],
        out_shape=[
            jax.ShapeDtypeStruct((n, d), jnp.float32),
            jax.ShapeDtypeStruct((n, d), jnp.float32),
        ],
    )(x, gamma.reshape(1, d), beta.reshape(1, d), bias.reshape(1, d), w_root)


def _combine_body(r_ref, p_ref, wn_ref, o_ref):
    agg = p_ref[0] + p_ref[1]
    o_ref[...] = r_ref[...] + jnp.dot(
        agg, wn_ref[...], preferred_element_type=jnp.float32)


def _combine(r, parts, w_nbr, blk):
    n, d = r.shape
    return pl.pallas_call(
        _combine_body,
        grid=(n // blk,),
        in_specs=[
            pl.BlockSpec((blk, d), lambda i: (i, 0)),
            pl.BlockSpec((2, blk, d), lambda i: (0, i, 0)),
            pl.BlockSpec((d, d), lambda i: (0, 0)),
        ],
        out_specs=pl.BlockSpec((blk, d), lambda i: (i, 0)),
        out_shape=jax.ShapeDtypeStruct((n, d), jnp.float32),
    )(r, parts, w_nbr)


def _sc_segment_sum(h, edges, n_pad, d, ng, gsz, bsz):
    rpt = n_pad // NS
    zb = bsz
    mesh = plsc.VectorSubcoreMesh(core_axis_name="c", subcore_axis_name="s")

    @functools.partial(
        pl.kernel,
        mesh=mesh,
        out_type=jax.ShapeDtypeStruct((NC, n_pad, d), jnp.float32),
        scratch_types=[
            pltpu.VMEM((2, gsz, bsz), jnp.int32),
            pltpu.VMEM((2, gsz, bsz), jnp.int32),
            pltpu.VMEM((2, bsz, d), jnp.float32),
            pltpu.VMEM_SHARED((n_pad, d), jnp.float32),
            pltpu.SemaphoreType.DMA,
            pltpu.SemaphoreType.DMA,
            pltpu.SemaphoreType.DMA,
        ],
    )
    def k(h_hbm, e_hbm, out_hbm, src_v, dst_v, rows_v, acc_s, gsem0, gsem1, isem):
        cid = lax.axis_index("c")
        sid = lax.axis_index("s")
        wid = cid * NS + sid

        d16 = d // 16

        def zstore(i, carry):
            r = i // d16
            c = (i % d16) * 16
            rows_v[0, r, pl.ds(c, 16)] = jnp.zeros((16,), jnp.float32)
            return carry

        lax.fori_loop(0, zb * d16, zstore, 0)

        zfull, zrem = divmod(rpt, zb)
        for i in range(zfull):
            pltpu.sync_copy(rows_v.at[0],
                            acc_s.at[pl.ds(sid * rpt + i * zb, zb)])
        if zrem:
            pltpu.sync_copy(rows_v.at[0, pl.ds(0, zrem)],
                            acc_s.at[pl.ds(sid * rpt + zfull * zb, zrem)])
        plsc.subcore_barrier()

        def gath(s, b, buf, sem):
            pltpu.async_copy(h_hbm.at[src_v.at[s, b]], rows_v.at[buf], sem)

        def gath_wait(s, b, buf, sem):
            pltpu.make_async_copy(h_hbm.at[src_v.at[s, b]], rows_v.at[buf], sem).wait()

        def scat(s, b, buf):
            pltpu.sync_copy(rows_v.at[buf], acc_s.at[dst_v.at[s, b]], add=True)

        pltpu.sync_copy(e_hbm.at[0, wid, 0], src_v.at[0])
        pltpu.sync_copy(e_hbm.at[1, wid, 0], dst_v.at[0])

        for g in range(ng):
            s = g % 2
            if g + 1 < ng:
                pltpu.async_copy(e_hbm.at[0, wid, g + 1], src_v.at[1 - s], isem)
                pltpu.async_copy(e_hbm.at[1, wid, g + 1], dst_v.at[1 - s], isem)
            gath(s, 0, 0, gsem0)

            def body(p, carry):
                b0 = 2 * p
                gath_wait(s, b0, 0, gsem0)
                gath(s, b0 + 1, 1, gsem1)
                scat(s, b0, 0)
                gath_wait(s, b0 + 1, 1, gsem1)
                gath(s, b0 + 2, 0, gsem0)
                scat(s, b0 + 1, 1)
                return carry

            lax.fori_loop(0, (gsz - 1) // 2, body, 0)
            gath_wait(s, gsz - 1, 0, gsem0)
            scat(s, gsz - 1, 0)
            if g + 1 < ng:
                pltpu.make_async_copy(e_hbm.at[0, wid, g + 1], src_v.at[1 - s], isem).wait()
                pltpu.make_async_copy(e_hbm.at[1, wid, g + 1], dst_v.at[1 - s], isem).wait()

        plsc.subcore_barrier()
        pltpu.sync_copy(acc_s.at[pl.ds(sid * rpt, rpt)],
                        out_hbm.at[cid, pl.ds(sid * rpt, rpt)])

    return k(h, edges)


def kernel(x, edge_index, W_nbr, W_root, b, gamma, beta):
    n, d = x.shape
    e = edge_index.shape[1]
    ept = e // NW
    bsz = 80
    gsz = 25
    ng = ept // (bsz * gsz)

    n_pad = ((n + 8 * NS - 1) // (8 * NS)) * (8 * NS)
    edges = edge_index.reshape(2, NW, ng, gsz, bsz)

    h, r = _ln_relu_root(x, gamma, beta, W_root, b, blk=10000)
    parts = _sc_segment_sum(h, edges, n_pad, d, ng, gsz, bsz)
    return _combine(r, parts, W_nbr, blk=10000)


# speedup vs baseline: 1.0220x; 1.0220x over previous
import functools

import jax
import jax.numpy as jnp
from jax import lax
from jax.experimental import pallas as pl
from jax.experimental.pallas import tpu as pltpu
from jax.experimental.pallas import tpu_sc as plsc

NC = 2
NS = 16
NW = NC * NS


def _ln_relu_body(x_ref, g_ref, bt_ref, bias_ref, wr_ref, h_ref, r_ref):
    xv = x_ref[...]
    mu = jnp.mean(xv, axis=1, keepdims=True)
    var = jnp.mean((xv - mu) ** 2, axis=1, keepdims=True)
    h = (xv - mu) * lax.rsqrt(var + 1e-5) * g_ref[...] + bt_ref[...]
    h = jnp.maximum(h, 0.0)
    h_ref[...] = h
    r_ref[...] = xv + bias_ref[...] + jnp.dot(
        h, wr_ref[...], preferred_element_type=jnp.float32)


def _ln_relu_root(x, gamma, beta, w_root, bias, blk):
    n, d = x.shape
    return pl.pallas_call(
        _ln_relu_body,
        grid=(n // blk,),
        in_specs=[
            pl.BlockSpec((blk, d), lambda i: (i, 0)),
            pl.BlockSpec((1, d), lambda i: (0, 0)),
            pl.BlockSpec((1, d), lambda i: (0, 0)),
            pl.BlockSpec((1, d), lambda i: (0, 0)),
            pl.BlockSpec((d, d), lambda i: (0, 0)),
        ],
        out_specs=[
            pl.BlockSpec((blk, d), lambda i: (i, 0)),
            pl.BlockSpec((blk, d), lambda i: (i, 0)),
        ],
        out_shape=[
            jax.ShapeDtypeStruct((n, d), jnp.float32),
            jax.ShapeDtypeStruct((n, d), jnp.float32),
        ],
    )(x, gamma.reshape(1, d), beta.reshape(1, d), bias.reshape(1, d), w_root)


def _combine_body(r_ref, p_ref, wn_ref, o_ref):
    agg = p_ref[0] + p_ref[1]
    o_ref[...] = r_ref[...] + jnp.dot(
        agg, wn_ref[...], preferred_element_type=jnp.float32)


def _combine(r, parts, w_nbr, blk):
    n, d = r.shape
    return pl.pallas_call(
        _combine_body,
        grid=(n // blk,),
        in_specs=[
            pl.BlockSpec((blk, d), lambda i: (i, 0)),
            pl.BlockSpec((2, blk, d), lambda i: (0, i, 0)),
            pl.BlockSpec((d, d), lambda i: (0, 0)),
        ],
        out_specs=pl.BlockSpec((blk, d), lambda i: (i, 0)),
        out_shape=jax.ShapeDtypeStruct((n, d), jnp.float32),
    )(r, parts, w_nbr)


def _sc_segment_sum(h, edges, n_pad, d, ng, gsz, bsz):
    rpt = n_pad // NS
    zb = bsz
    mesh = plsc.VectorSubcoreMesh(core_axis_name="c", subcore_axis_name="s")

    @functools.partial(
        pl.kernel,
        mesh=mesh,
        out_type=jax.ShapeDtypeStruct((NC, n_pad, d), jnp.float32),
        scratch_types=[
            pltpu.VMEM((2, gsz, bsz), jnp.int32),
            pltpu.VMEM((2, gsz, bsz), jnp.int32),
            pltpu.VMEM((2, bsz, d), jnp.float32),
            pltpu.VMEM_SHARED((n_pad, d), jnp.float32),
            pltpu.SemaphoreType.DMA,
            pltpu.SemaphoreType.DMA,
            pltpu.SemaphoreType.DMA,
        ],
    )
    def k(h_hbm, e_hbm, out_hbm, src_v, dst_v, rows_v, acc_s, gsem0, gsem1, isem):
        cid = lax.axis_index("c")
        sid = lax.axis_index("s")
        wid = cid * NS + sid

        d16 = d // 16

        def zstore(i, carry):
            for c in range(d16):
                rows_v[0, i, pl.ds(c * 16, 16)] = jnp.zeros((16,), jnp.float32)
            return carry

        lax.fori_loop(0, zb, zstore, 0)

        zfull, zrem = divmod(rpt, zb)
        for i in range(zfull):
            pltpu.async_copy(rows_v.at[0],
                             acc_s.at[pl.ds(sid * rpt + i * zb, zb)], gsem0)
        if zrem:
            pltpu.async_copy(rows_v.at[0, pl.ds(0, zrem)],
                             acc_s.at[pl.ds(sid * rpt + zfull * zb, zrem)], gsem1)
        for i in range(zfull):
            pltpu.make_async_copy(rows_v.at[0],
                                  acc_s.at[pl.ds(sid * rpt + i * zb, zb)],
                                  gsem0).wait()
        if zrem:
            pltpu.make_async_copy(rows_v.at[0, pl.ds(0, zrem)],
                                  acc_s.at[pl.ds(sid * rpt + zfull * zb, zrem)],
                                  gsem1).wait()
        plsc.subcore_barrier()

        def gath(s, b, buf, sem):
            pltpu.async_copy(h_hbm.at[src_v.at[s, b]], rows_v.at[buf], sem)

        def gath_wait(s, b, buf, sem):
            pltpu.make_async_copy(h_hbm.at[src_v.at[s, b]], rows_v.at[buf], sem).wait()

        def scat(s, b, buf):
            pltpu.sync_copy(rows_v.at[buf], acc_s.at[dst_v.at[s, b]], add=True)

        pltpu.sync_copy(e_hbm.at[0, wid, 0], src_v.at[0])
        pltpu.sync_copy(e_hbm.at[1, wid, 0], dst_v.at[0])

        for g in range(ng):
            s = g % 2
            if g + 1 < ng:
                pltpu.async_copy(e_hbm.at[0, wid, g + 1], src_v.at[1 - s], isem)
                pltpu.async_copy(e_hbm.at[1, wid, g + 1], dst_v.at[1 - s], isem)
            gath(s, 0, 0, gsem0)

            def body(p, carry):
                b0 = 2 * p
                gath_wait(s, b0, 0, gsem0)
                gath(s, b0 + 1, 1, gsem1)
                scat(s, b0, 0)
                gath_wait(s, b0 + 1, 1, gsem1)
                gath(s, b0 + 2, 0, gsem0)
                scat(s, b0 + 1, 1)
                return carry

            lax.fori_loop(0, (gsz - 1) // 2, body, 0)
            gath_wait(s, gsz - 1, 0, gsem0)
            scat(s, gsz - 1, 0)
            if g + 1 < ng:
                pltpu.make_async_copy(e_hbm.at[0, wid, g + 1], src_v.at[1 - s], isem).wait()
                pltpu.make_async_copy(e_hbm.at[1, wid, g + 1], dst_v.at[1 - s], isem).wait()

        plsc.subcore_barrier()
        pltpu.sync_copy(acc_s.at[pl.ds(sid * rpt, rpt)],
                        out_hbm.at[cid, pl.ds(sid * rpt, rpt)])

    return k(h, edges)


def kernel(x, edge_index, W_nbr, W_root, b, gamma, beta):
    n, d = x.shape
    e = edge_index.shape[1]
    ept = e // NW
    bsz = 80
    gsz = 25
    ng = ept // (bsz * gsz)

    n_pad = ((n + 8 * NS - 1) // (8 * NS)) * (8 * NS)
    edges = edge_index.reshape(2, NW, ng, gsz, bsz)

    h, r = _ln_relu_root(x, gamma, beta, W_root, b, blk=5000)
    parts = _sc_segment_sum(h, edges, n_pad, d, ng, gsz, bsz)
    return _combine(r, parts, W_nbr, blk=5000)
